# baseline (speedup 1.0000x reference)
import functools

import jax
import jax.numpy as jnp
from jax import lax
from jax.experimental import pallas as pl
from jax.experimental.pallas import tpu as pltpu
from jax.experimental.pallas import tpu_sc as plsc

F32 = jnp.float32
TW = 256
PW = 144
NBLK = 1000
EBLK = 640


def _silu(v):
    return v / (1.0 + jnp.exp(-v))


def _dot(a, b):
    return jnp.dot(a, b, preferred_element_type=F32)


def _pre_body(x_ref, p8_ref, wemb_ref, bemb_ref, wd_ref, ws_ref,
              h_ref, td_ref, ts_ref):
    h = _dot(x_ref[...], wemb_ref[...]) + bemb_ref[...]
    h_ref[...] = h
    p8 = p8_ref[...]
    z = jnp.zeros((h.shape[0], TW - 136), F32)
    td_ref[...] = jnp.concatenate([_dot(h, wd_ref[...]), p8, z], axis=1)
    ts_ref[...] = jnp.concatenate([_dot(h, ws_ref[...]), p8, z], axis=1)


_SEL48 = None


def _edge_body(u_ref, xp_ref, ea_ref, wex_ref, be1_ref, we2_ref, be2_ref,
               wx1_ref, bx1_ref, wx2_ref, bx2_ref, m_ref, aux_ref):
    u = u_ref[...]
    relp = xp_ref[...][:, :8]
    d2 = jnp.sum(relp * relp, axis=1, keepdims=True)
    extra = jnp.concatenate([d2, ea_ref[...]], axis=1)
    m1 = _silu(u + _dot(extra, wex_ref[...]) + be1_ref[...])
    m = _silu(_dot(m1, we2_ref[...]) + be2_ref[...])
    t1 = _silu(_dot(m, wx1_ref[...]) + bx1_ref[...])
    xw = jnp.sum(t1 * wx2_ref[...], axis=1, keepdims=True) + bx2_ref[...]
    m_ref[...] = m
    rx = relp * xw
    sel = jnp.concatenate(
        [jnp.eye(3, 8, dtype=F32), jnp.zeros((1, 8), F32)], axis=0)
    aux = lax.dot_general(sel, rx, (((1,), (1,)), ((), ())),
                          preferred_element_type=F32)
    aux_ref[...] = aux + jnp.concatenate(
        [jnp.zeros((3, aux.shape[1]), F32), jnp.ones((1, aux.shape[1]), F32)], axis=0)


def _node_body(a_ref, x_ref, h_ref, p8_ref, wh1a_ref,
               wh1b_ref, bh1_ref, wh2_ref, bh2_ref, wd_ref, ws_ref,
               hn_ref, pn_ref, td_ref, ts_ref):
    aggm = a_ref[0]
    small = x_ref[0]
    deg = small[:, 3:4]
    n = small.shape[0]
    aggx = jnp.concatenate([small[:, :3], jnp.zeros((n, 5), F32)], axis=1)
    p_new = p8_ref[...] + aggx / (deg + 1.0)
    h = h_ref[...]
    hu = _silu(_dot(h, wh1a_ref[...]) + _dot(aggm, wh1b_ref[...]) + bh1_ref[...])
    h_new = h + _dot(hu, wh2_ref[...]) + bh2_ref[...]
    hn_ref[...] = h_new
    pn_ref[...] = p_new
    if td_ref is not None:
        z = jnp.zeros((h.shape[0], TW - 136), F32)
        td_ref[...] = jnp.concatenate([_dot(h_new, wd_ref[...]), p_new, z], axis=1)
        ts_ref[...] = jnp.concatenate([_dot(h_new, ws_ref[...]), p_new, z], axis=1)


def _node_last_body(a_ref, h_ref, wh1a_ref, wh1b_ref, bh1_ref,
                    wh2_ref, bh2_ref, hn_ref):
    aggm = a_ref[0]
    h = h_ref[...]
    hu = _silu(_dot(h, wh1a_ref[...]) + _dot(aggm, wh1b_ref[...]) + bh1_ref[...])
    hn_ref[...] = h + _dot(hu, wh2_ref[...]) + bh2_ref[...]


def _pool_body(h_ref, bf_ref, g_ref):
    i = pl.program_id(0)

    @pl.when(i == 0)
    def _():
        g_ref[...] = jnp.zeros_like(g_ref)

    n = h_ref.shape[0]
    bf = bf_ref[0]
    rows = lax.broadcasted_iota(jnp.int32, (128, n), 0).astype(F32)
    onehot = jnp.where(rows == bf, 1.0, 0.0)
    hb = jnp.concatenate([h_ref[...], jnp.ones((n, 16), F32)], axis=1)
    g_ref[...] += _dot(onehot, hb)


def _head_body(g_ref, tid_ref, wha_ref, bha_ref, whb_ref, bhb_ref, out_ref):
    ga = g_ref[...]
    cnt = jnp.maximum(ga[:, 128:129], 1.0)
    g = ga[:, :128] / cnt
    tid = tid_ref[...]
    nt = wha_ref.shape[0]
    logits = jnp.zeros((g.shape[0], 1), F32)
    for t in range(nt):
        hid = _silu(_dot(g, wha_ref[t]) + bha_ref[t][None, :])
        o = jnp.sum(hid * whb_ref[t][None, :], axis=1, keepdims=True) + bhb_ref[t, 0]
        logits = jnp.where(tid == t, o, logits)
    out_ref[...] = logits


def _sc_mesh():
    return plsc.VectorSubcoreMesh(core_axis_name="c", subcore_axis_name="s")


def _make_gather(E):
    nch = E // 64
    tmax = (nch + 63) // 64

    def _compute(db, sb, ub, xb):
        @pl.loop(0, 64, unroll=8)
        def _(r):
            for c in range(8):
                sl = pl.ds(c * 16, 16)
                ub[r, sl] = db[r, sl] + sb[r, sl]
            pp = pl.ds(128, 16)
            xb[r, :] = db[r, pp] - sb[r, pp]

    @functools.partial(
        pl.kernel,
        out_type=(jax.ShapeDtypeStruct((E, 128), F32),
                  jax.ShapeDtypeStruct((E, 16), F32)),
        mesh=_sc_mesh(),
        scratch_types=[
            pltpu.VMEM((64,), jnp.int32), pltpu.VMEM((64,), jnp.int32),
            pltpu.VMEM((64,), jnp.int32), pltpu.VMEM((64,), jnp.int32),
            pltpu.VMEM((64, TW), F32), pltpu.VMEM((64, TW), F32),
            pltpu.VMEM((64, TW), F32), pltpu.VMEM((64, TW), F32),
            pltpu.VMEM((64, 128), F32), pltpu.VMEM((64, 128), F32),
            pltpu.VMEM((64, 16), F32), pltpu.VMEM((64, 16), F32),
            pltpu.SemaphoreType.DMA, pltpu.SemaphoreType.DMA,
            pltpu.SemaphoreType.DMA, pltpu.SemaphoreType.DMA,
        ],
    )
    def gath(tbl_d, tbl_s, dst2, src2, out_u, out_x,
             di_a, si_a, di_b, si_b, db_a, sb_a, db_b, sb_b,
             ub_a, ub_b, xb_a, xb_b, sg_a, sg_b, sw_a, sw_b):
        wid = lax.axis_index("s") * 2 + lax.axis_index("c")

        def stage(ci, di, si, db, sb, sg):
            @pl.when(ci < nch)
            def _():
                pltpu.sync_copy(dst2.at[ci], di)
                pltpu.sync_copy(src2.at[ci], si)
                pltpu.async_copy(tbl_d.at[di], db, sg)
                pltpu.async_copy(tbl_s.at[si], sb, sg)

        def consume(ci, db, sb, ub, xb, sg, sw):
            @pl.when(ci < nch)
            def _():
                pltpu.make_async_copy(tbl_d.at[pl.ds(0, 64)], db, sg).wait()
                pltpu.make_async_copy(tbl_s.at[pl.ds(0, 64)], sb, sg).wait()

                @pl.when(ci >= wid + 64)
                def _():
                    pltpu.make_async_copy(out_u.at[pl.ds(0, 64)], ub, sw).wait()
                    pltpu.make_async_copy(out_x.at[pl.ds(0, 64)], xb, sw).wait()

                _compute(db, sb, ub, xb)
                pltpu.async_copy(ub, out_u.at[pl.ds(ci * 64, 64)], sw)
                pltpu.async_copy(xb, out_x.at[pl.ds(ci * 64, 64)], sw)

        stage(wid, di_a, si_a, db_a, sb_a, sg_a)

        @pl.loop(0, tmax)
        def _(t):
            ci0 = wid + 64 * t
            ci1 = ci0 + 32
            stage(ci1, di_b, si_b, db_b, sb_b, sg_b)
            consume(ci0, db_a, sb_a, ub_a, xb_a, sg_a, sw_a)
            stage(ci0 + 64, di_a, si_a, db_a, sb_a, sg_a)
            consume(ci1, db_b, sb_b, ub_b, xb_b, sg_b, sw_b)

        pltpu.make_async_copy(out_u.at[pl.ds(0, 64)], ub_a, sw_a).wait()
        pltpu.make_async_copy(out_x.at[pl.ds(0, 64)], xb_a, sw_a).wait()
        pltpu.make_async_copy(out_u.at[pl.ds(0, 64)], ub_b, sw_b).wait()
        pltpu.make_async_copy(out_x.at[pl.ds(0, 64)], xb_b, sw_b).wait()

    return gath


def _make_scatter(E, N):
    nch = E // 128
    hn = N // 2
    hnp = ((hn + 64 + 127) // 128) * 128
    rpt = hnp // 16

    @functools.partial(
        pl.kernel,
        out_type=(jax.ShapeDtypeStruct((2 * hnp, 128), F32),
                  jax.ShapeDtypeStruct((2 * hnp * 4,), F32)),
        mesh=_sc_mesh(),
        scratch_types=[
            pltpu.VMEM((128,), jnp.int32), pltpu.VMEM((128,), jnp.int32),
            pltpu.VMEM((128,), jnp.int32), pltpu.VMEM((128,), jnp.int32),
            pltpu.VMEM((128, 128), F32), pltpu.VMEM((128,), F32),
            pltpu.VMEM((hnp * 4,), F32),
            pltpu.VMEM_SHARED((hnp, 128), F32),
            pltpu.VMEM_SHARED((hnp * 4,), F32),
        ],
    )
    def scat(m_rows, aux3, dst2, zeros_nw, zeros_x, out_m, out_x,
             di_v, mi_v, xb_i, xi_v, mb_v, xb_v, xd_v, acc_sh, acx_sh):
        c0 = lax.axis_index("c")
        s0 = lax.axis_index("s")
        base = s0 * rpt
        lo = c0 * hn

        pltpu.sync_copy(zeros_nw.at[pl.ds(base, rpt)],
                        acc_sh.at[pl.ds(base, rpt)])

        @pl.when(s0 == 0)
        def _():
            pltpu.sync_copy(zeros_x, xd_v)
            pltpu.sync_copy(xd_v, acx_sh)

        plsc.subcore_barrier()

        @pl.loop(s0, nch, step=16)
        def _(ci):
            pltpu.sync_copy(dst2.at[ci], di_v)
            pltpu.sync_copy(m_rows.at[pl.ds(ci * 128, 128)], mb_v)
            for j in range(8):
                sl = pl.ds(j * 16, 16)
                di = di_v[sl]
                off = di - lo
                ok = (off >= 0) & (off < hn)
                mi_v[sl] = jnp.where(ok, off, hn + (di & 63))
                xb_i[sl] = jnp.where(ok, off * 4, hn * 4 + (di & 255))
            pltpu.sync_copy(mb_v, acc_sh.at[mi_v], add=True)
            for k in range(4):
                for j in range(8):
                    sl = pl.ds(j * 16, 16)
                    xi_v[sl] = xb_i[sl] + k
                pltpu.sync_copy(aux3.at[k, ci], xb_v)
                pltpu.sync_copy(xb_v, acx_sh.at[xi_v], add=True)

        plsc.subcore_barrier()

        pltpu.sync_copy(acc_sh.at[pl.ds(base, rpt)],
                        out_m.at[pl.ds(c0 * hnp + base, rpt)])

        @pl.when(s0 == 1)
        def _():
            pltpu.sync_copy(acx_sh, xd_v)
            pltpu.sync_copy(xd_v, out_x.at[pl.ds(c0 * hnp * 4, hnp * 4)])

    return scat


def kernel(x, pos, edge_attr, edge_index, batch_idx, task_id, Wemb, bemb,
           We1, be1, We2, be2, Wx1, bx1, Wx2, bx2, Wh1, bh1, Wh2, bh2,
           Wha, bha, Whb, bhb):
    N, ND = x.shape
    E, ED = edge_attr.shape
    B = task_id.shape[0]
    H = Wemb.shape[1]
    L = We1.shape[0]

    src2g = edge_index[0].reshape(E // 64, 64)
    dst2g = edge_index[1].reshape(E // 64, 64)
    dst2 = edge_index[1].reshape(E // 128, 128)
    p8 = jnp.pad(pos, ((0, 0), (0, 8 - pos.shape[1])))
    batch_f = batch_idx.astype(F32).reshape(N // NBLK, 1, NBLK)
    tid2 = task_id.reshape(B, 1)
    hn = N // 2
    hnp = ((hn + 64 + 127) // 128) * 128
    zeros_nw = jnp.zeros((N, 128), F32)
    zeros_x = jnp.zeros((hnp * 4,), F32)

    w1d = We1[:, :H, :]
    w1s = We1[:, H:2 * H, :]
    w1x = We1[:, 2 * H:, :]
    wh1a = Wh1[:, :H, :]
    wh1b = Wh1[:, H:, :]
    wx2r = Wx2.reshape(L, 1, H)
    whbr = Whb.reshape(Whb.shape[0], Whb.shape[1])

    gn = N // NBLK
    ge = E // EBLK

    full = lambda shape: pl.BlockSpec(shape, lambda *_: tuple(0 for _ in shape))
    rowsN = lambda w: pl.BlockSpec((NBLK, w), lambda i: (i, 0))
    rowsE = lambda w: pl.BlockSpec((EBLK, w), lambda i: (i, 0))

    h, tbl_d, tbl_s = pl.pallas_call(
        _pre_body,
        grid=(gn,),
        in_specs=[rowsN(ND), rowsN(8), full((ND, H)), full((1, H)),
                  full((H, H)), full((H, H))],
        out_specs=[rowsN(H), rowsN(TW), rowsN(TW)],
        out_shape=[jax.ShapeDtypeStruct((N, H), F32),
                   jax.ShapeDtypeStruct((N, TW), F32),
                   jax.ShapeDtypeStruct((N, TW), F32)],
    )(x, p8, Wemb, bemb.reshape(1, H), w1d[0], w1s[0])

    gather = _make_gather(E)
    scatter = _make_scatter(E, N)

    p8_cur = p8
    for l in range(L):
        gu, gx = gather(tbl_d, tbl_s, dst2g, src2g)

        m_rows, aux = pl.pallas_call(
            _edge_body,
            grid=(ge,),
            in_specs=[rowsE(128), rowsE(16), rowsE(ED), full((1 + ED, H)),
                      full((1, H)), full((H, H)), full((1, H)),
                      full((H, H)), full((1, H)), full((1, H)), full((1, 1))],
            out_specs=[rowsE(128), pl.BlockSpec((4, EBLK), lambda i: (0, i))],
            out_shape=[jax.ShapeDtypeStruct((E, 128), F32),
                       jax.ShapeDtypeStruct((4, E), F32)],
        )(gu, gx, edge_attr, w1x[l], be1[l].reshape(1, H), We2[l],
          be2[l].reshape(1, H), Wx1[l], bx1[l].reshape(1, H), wx2r[l],
          bx2[l].reshape(1, 1))

        acc_m, acc_x = scatter(m_rows, aux.reshape(4, E // 128, 128), dst2,
                               zeros_nw, zeros_x)
        accm3 = acc_m.reshape(2, hnp, 128)
        accx3 = acc_x.reshape(2, hnp, 4)

        hb = gn // 2
        a_spec = pl.BlockSpec((1, NBLK, 128), lambda i: (i // hb, i % hb, 0))
        x_spec = pl.BlockSpec((1, NBLK, 4), lambda i: (i // hb, i % hb, 0))
        if l < L - 1:
            h, p8_cur, tbl_d, tbl_s = pl.pallas_call(
                _node_body,
                grid=(gn,),
                in_specs=[a_spec, x_spec, rowsN(H), rowsN(8),
                          full((H, H)), full((H, H)), full((1, H)),
                          full((H, H)), full((1, H)),
                          full((H, H)), full((H, H))],
                out_specs=[rowsN(H), rowsN(8), rowsN(TW), rowsN(TW)],
                out_shape=[jax.ShapeDtypeStruct((N, H), F32),
                           jax.ShapeDtypeStruct((N, 8), F32),
                           jax.ShapeDtypeStruct((N, TW), F32),
                           jax.ShapeDtypeStruct((N, TW), F32)],
            )(accm3, accx3, h, p8_cur, wh1a[l], wh1b[l],
              bh1[l].reshape(1, H), Wh2[l], bh2[l].reshape(1, H),
              w1d[l + 1], w1s[l + 1])
        else:
            h = pl.pallas_call(
                _node_last_body,
                grid=(gn,),
                in_specs=[a_spec, rowsN(H),
                          full((H, H)), full((H, H)), full((1, H)),
                          full((H, H)), full((1, H))],
                out_specs=rowsN(H),
                out_shape=jax.ShapeDtypeStruct((N, H), F32),
            )(accm3, h, wh1a[l], wh1b[l], bh1[l].reshape(1, H),
              Wh2[l], bh2[l].reshape(1, H))

    g_aug = pl.pallas_call(
        _pool_body,
        grid=(gn,),
        in_specs=[rowsN(H), pl.BlockSpec((1, 1, NBLK), lambda i: (i, 0, 0))],
        out_specs=pl.BlockSpec((B, PW), lambda i: (0, 0)),
        out_shape=jax.ShapeDtypeStruct((B, PW), F32),
    )(h, batch_f)

    logits = pl.pallas_call(
        _head_body,
        in_specs=[full((B, PW)), full((B, 1)), full(Wha.shape), full(bha.shape),
                  full(whbr.shape), full(bhb.shape)],
        out_specs=full((B, 1)),
        out_shape=jax.ShapeDtypeStruct((B, 1), F32),
    )(g_aug, tid2, Wha, bha, whbr, bhb)

    return logits


# speedup vs baseline: 1.3207x; 1.3207x over previous
import functools

import jax
import jax.numpy as jnp
from jax import lax
from jax.experimental import pallas as pl
from jax.experimental.pallas import tpu as pltpu
from jax.experimental.pallas import tpu_sc as plsc

F32 = jnp.float32
TW = 256
PW = 144
NBLK = 1000
EBLK = 640


def _silu(v):
    return v / (1.0 + jnp.exp(-v))


def _dot(a, b):
    return jnp.dot(a, b, preferred_element_type=F32)


def _pre_body(x_ref, p8_ref, wemb_ref, bemb_ref, wd_ref, ws_ref,
              h_ref, td_ref, ts_ref):
    h = _dot(x_ref[...], wemb_ref[...]) + bemb_ref[...]
    h_ref[...] = h
    p8 = p8_ref[...]
    z = jnp.zeros((h.shape[0], TW - 136), F32)
    td_ref[...] = jnp.concatenate([_dot(h, wd_ref[...]), p8, z], axis=1)
    ts_ref[...] = jnp.concatenate([_dot(h, ws_ref[...]), p8, z], axis=1)


_SEL48 = None


def _edge_body(u_ref, xp_ref, ea_ref, wex_ref, be1_ref, we2_ref, be2_ref,
               wx1_ref, bx1_ref, wx2_ref, bx2_ref, m_ref, aux_ref):
    u = u_ref[...]
    relp = xp_ref[...][:, :8]
    d2 = jnp.sum(relp * relp, axis=1, keepdims=True)
    extra = jnp.concatenate([d2, ea_ref[...]], axis=1)
    m1 = _silu(u + _dot(extra, wex_ref[...]) + be1_ref[...])
    m = _silu(_dot(m1, we2_ref[...]) + be2_ref[...])
    t1 = _silu(_dot(m, wx1_ref[...]) + bx1_ref[...])
    xw = jnp.sum(t1 * wx2_ref[...], axis=1, keepdims=True) + bx2_ref[...]
    m_ref[...] = m
    rx = relp * xw
    sel = jnp.concatenate(
        [jnp.eye(3, 8, dtype=F32), jnp.zeros((1, 8), F32)], axis=0)
    aux = lax.dot_general(sel, rx, (((1,), (1,)), ((), ())),
                          preferred_element_type=F32)
    aux_ref[...] = aux + jnp.concatenate(
        [jnp.zeros((3, aux.shape[1]), F32), jnp.ones((1, aux.shape[1]), F32)], axis=0)


def _node_body(a_ref, x_ref, h_ref, p8_ref, wh1a_ref,
               wh1b_ref, bh1_ref, wh2_ref, bh2_ref, wd_ref, ws_ref,
               hn_ref, pn_ref, td_ref, ts_ref):
    aggm = a_ref[0]
    small = x_ref[0]
    deg = small[:, 3:4]
    n = small.shape[0]
    aggx = jnp.concatenate([small[:, :3], jnp.zeros((n, 5), F32)], axis=1)
    p_new = p8_ref[...] + aggx / (deg + 1.0)
    h = h_ref[...]
    hu = _silu(_dot(h, wh1a_ref[...]) + _dot(aggm, wh1b_ref[...]) + bh1_ref[...])
    h_new = h + _dot(hu, wh2_ref[...]) + bh2_ref[...]
    hn_ref[...] = h_new
    pn_ref[...] = p_new
    if td_ref is not None:
        z = jnp.zeros((h.shape[0], TW - 136), F32)
        td_ref[...] = jnp.concatenate([_dot(h_new, wd_ref[...]), p_new, z], axis=1)
        ts_ref[...] = jnp.concatenate([_dot(h_new, ws_ref[...]), p_new, z], axis=1)


def _node_last_body(a_ref, h_ref, wh1a_ref, wh1b_ref, bh1_ref,
                    wh2_ref, bh2_ref, hn_ref):
    aggm = a_ref[0]
    h = h_ref[...]
    hu = _silu(_dot(h, wh1a_ref[...]) + _dot(aggm, wh1b_ref[...]) + bh1_ref[...])
    hn_ref[...] = h + _dot(hu, wh2_ref[...]) + bh2_ref[...]


def _pool_body(h_ref, bf_ref, g_ref):
    i = pl.program_id(0)

    @pl.when(i == 0)
    def _():
        g_ref[...] = jnp.zeros_like(g_ref)

    n = h_ref.shape[0]
    bf = bf_ref[0]
    rows = lax.broadcasted_iota(jnp.int32, (128, n), 0).astype(F32)
    onehot = jnp.where(rows == bf, 1.0, 0.0)
    hb = jnp.concatenate([h_ref[...], jnp.ones((n, 16), F32)], axis=1)
    g_ref[...] += _dot(onehot, hb)


def _head_body(g_ref, tid_ref, wha_ref, bha_ref, whb_ref, bhb_ref, out_ref):
    ga = g_ref[...]
    cnt = jnp.maximum(ga[:, 128:129], 1.0)
    g = ga[:, :128] / cnt
    tid = tid_ref[...]
    nt = wha_ref.shape[0]
    logits = jnp.zeros((g.shape[0], 1), F32)
    for t in range(nt):
        hid = _silu(_dot(g, wha_ref[t]) + bha_ref[t][None, :])
        o = jnp.sum(hid * whb_ref[t][None, :], axis=1, keepdims=True) + bhb_ref[t, 0]
        logits = jnp.where(tid == t, o, logits)
    out_ref[...] = logits


def _sc_mesh():
    return plsc.VectorSubcoreMesh(core_axis_name="c", subcore_axis_name="s")


def _make_gather(E):
    nch = E // 64
    tmax = (nch + 63) // 64

    def _compute(db, sb, ub, xb):
        @pl.loop(0, 64, unroll=8)
        def _(r):
            for c in range(8):
                sl = pl.ds(c * 16, 16)
                ub[r, sl] = db[r, sl] + sb[r, sl]
            pp = pl.ds(128, 16)
            xb[r, :] = db[r, pp] - sb[r, pp]

    @functools.partial(
        pl.kernel,
        out_type=(jax.ShapeDtypeStruct((E, 128), F32),
                  jax.ShapeDtypeStruct((E, 16), F32)),
        mesh=_sc_mesh(),
        scratch_types=[
            pltpu.VMEM((64,), jnp.int32), pltpu.VMEM((64,), jnp.int32),
            pltpu.VMEM((64,), jnp.int32), pltpu.VMEM((64,), jnp.int32),
            pltpu.VMEM((64, TW), F32), pltpu.VMEM((64, TW), F32),
            pltpu.VMEM((64, TW), F32), pltpu.VMEM((64, TW), F32),
            pltpu.VMEM((64, 128), F32), pltpu.VMEM((64, 128), F32),
            pltpu.VMEM((64, 16), F32), pltpu.VMEM((64, 16), F32),
            pltpu.SemaphoreType.DMA, pltpu.SemaphoreType.DMA,
            pltpu.SemaphoreType.DMA, pltpu.SemaphoreType.DMA,
        ],
    )
    def gath(tbl_d, tbl_s, dst2, src2, out_u, out_x,
             di_a, si_a, di_b, si_b, db_a, sb_a, db_b, sb_b,
             ub_a, ub_b, xb_a, xb_b, sg_a, sg_b, sw_a, sw_b):
        wid = lax.axis_index("s") * 2 + lax.axis_index("c")

        def stage(ci, di, si, db, sb, sg):
            @pl.when(ci < nch)
            def _():
                pltpu.sync_copy(dst2.at[ci], di)
                pltpu.sync_copy(src2.at[ci], si)
                pltpu.async_copy(tbl_d.at[di], db, sg)
                pltpu.async_copy(tbl_s.at[si], sb, sg)

        def consume(ci, db, sb, ub, xb, sg, sw):
            @pl.when(ci < nch)
            def _():
                pltpu.make_async_copy(tbl_d.at[pl.ds(0, 64)], db, sg).wait()
                pltpu.make_async_copy(tbl_s.at[pl.ds(0, 64)], sb, sg).wait()

                @pl.when(ci >= wid + 64)
                def _():
                    pltpu.make_async_copy(out_u.at[pl.ds(0, 64)], ub, sw).wait()
                    pltpu.make_async_copy(out_x.at[pl.ds(0, 64)], xb, sw).wait()

                _compute(db, sb, ub, xb)
                pltpu.async_copy(ub, out_u.at[pl.ds(ci * 64, 64)], sw)
                pltpu.async_copy(xb, out_x.at[pl.ds(ci * 64, 64)], sw)

        stage(wid, di_a, si_a, db_a, sb_a, sg_a)

        @pl.loop(0, tmax)
        def _(t):
            ci0 = wid + 64 * t
            ci1 = ci0 + 32
            stage(ci1, di_b, si_b, db_b, sb_b, sg_b)
            consume(ci0, db_a, sb_a, ub_a, xb_a, sg_a, sw_a)
            stage(ci0 + 64, di_a, si_a, db_a, sb_a, sg_a)
            consume(ci1, db_b, sb_b, ub_b, xb_b, sg_b, sw_b)

        pltpu.make_async_copy(out_u.at[pl.ds(0, 64)], ub_a, sw_a).wait()
        pltpu.make_async_copy(out_x.at[pl.ds(0, 64)], xb_a, sw_a).wait()
        pltpu.make_async_copy(out_u.at[pl.ds(0, 64)], ub_b, sw_b).wait()
        pltpu.make_async_copy(out_x.at[pl.ds(0, 64)], xb_b, sw_b).wait()

    return gath


def _make_scatter(E, N):
    nch = E // 128
    hn = N // 2
    hnp = ((hn + 64 + 127) // 128) * 128
    rpt = hnp // 16
    hnp4 = hnp * 4
    xsl = hnp4 // 16

    @functools.partial(
        pl.kernel,
        out_type=(jax.ShapeDtypeStruct((2 * hnp, 128), F32),
                  jax.ShapeDtypeStruct((2 * hnp4,), F32)),
        mesh=_sc_mesh(),
        scratch_types=[
            pltpu.VMEM((128,), jnp.int32), pltpu.VMEM((128,), jnp.int32),
            pltpu.VMEM((128,), jnp.int32),
            pltpu.VMEM((4, 128), jnp.int32), pltpu.VMEM((4, 128), jnp.int32),
            pltpu.VMEM((128, 128), F32), pltpu.VMEM((128, 128), F32),
            pltpu.VMEM((4, 128), F32), pltpu.VMEM((4, 128), F32),
            pltpu.VMEM((xsl,), F32),
            pltpu.SemaphoreType.DMA, pltpu.SemaphoreType.DMA,
            pltpu.SemaphoreType.DMA, pltpu.SemaphoreType.DMA,
            pltpu.VMEM_SHARED((hnp, 128), F32),
            pltpu.VMEM_SHARED((hnp4,), F32),
        ],
    )
    def scat(m_rows, aux_t, dst2, zeros_nw, zeros_x, out_m, out_x,
             di_a, di_b, mi_v, xk_a, xk_b, mb_a, mb_b, xv_a, xv_b, red_v,
             sm_a, sm_b, sx_a, sx_b, acc_sh, acx_sh):
        c0 = lax.axis_index("c")
        s0 = lax.axis_index("s")
        base = s0 * rpt
        lo = c0 * hn
        xbase = s0 * xsl

        pltpu.sync_copy(zeros_nw.at[pl.ds(base, rpt)],
                        acc_sh.at[pl.ds(base, rpt)])
        pltpu.sync_copy(zeros_x.at[pl.ds(xbase, xsl)], red_v)
        pltpu.sync_copy(red_v, acx_sh.at[pl.ds(xbase, xsl)])
        plsc.subcore_barrier()

        def stage(ci, di, mb, xv, sm):
            @pl.when(ci < nch)
            def _():
                pltpu.async_copy(dst2.at[ci], di, sm)
                pltpu.async_copy(m_rows.at[pl.ds(ci * 128, 128)], mb, sm)
                pltpu.async_copy(aux_t.at[ci], xv, sm)

        def consume(ci, di, mb, xv, xk, sm, sx):
            @pl.when(ci < nch)
            def _():
                pltpu.make_async_copy(dst2.at[0], di, sm).wait()
                pltpu.make_async_copy(m_rows.at[pl.ds(0, 128)], mb, sm).wait()
                pltpu.make_async_copy(aux_t.at[0], xv, sm).wait()

                @pl.when(ci >= s0 + 32)
                def _():
                    for k in range(4):
                        pltpu.make_async_copy(aux_t.at[0, 0], xv.at[0],
                                              sx).wait()

                for j in range(8):
                    sl = pl.ds(j * 16, 16)
                    di16 = di[sl]
                    off = di16 - lo
                    ok = (off >= 0) & (off < hn)
                    mi_v[sl] = jnp.where(ok, off, hn + (di16 & 63))
                    xb = jnp.where(ok, off * 4, hn * 4 + (di16 & 255))
                    for k in range(4):
                        xk[k, sl] = xb + k
                pltpu.sync_copy(mb, acc_sh.at[mi_v], add=True)
                for k in range(4):
                    pltpu.async_copy(xv.at[k], acx_sh.at[xk.at[k]], sx,
                                     add=True)

        stage(s0, di_a, mb_a, xv_a, sm_a)

        @pl.loop(0, (nch + 31) // 32)
        def _(t):
            ci0 = s0 + 32 * t
            ci1 = ci0 + 16
            stage(ci1, di_b, mb_b, xv_b, sm_b)
            consume(ci0, di_a, mb_a, xv_a, xk_a, sm_a, sx_a)
            stage(ci0 + 32, di_a, mb_a, xv_a, sm_a)
            consume(ci1, di_b, mb_b, xv_b, xk_b, sm_b, sx_b)

        for k in range(4):
            pltpu.make_async_copy(aux_t.at[0, 0], xv_a.at[0], sx_a).wait()
            pltpu.make_async_copy(aux_t.at[0, 0], xv_b.at[0], sx_b).wait()
        plsc.subcore_barrier()

        pltpu.sync_copy(acc_sh.at[pl.ds(base, rpt)],
                        out_m.at[pl.ds(c0 * hnp + base, rpt)])
        pltpu.sync_copy(acx_sh.at[pl.ds(xbase, xsl)], red_v)
        pltpu.sync_copy(red_v, out_x.at[pl.ds(c0 * hnp4 + xbase, xsl)])

    return scat


def kernel(x, pos, edge_attr, edge_index, batch_idx, task_id, Wemb, bemb,
           We1, be1, We2, be2, Wx1, bx1, Wx2, bx2, Wh1, bh1, Wh2, bh2,
           Wha, bha, Whb, bhb):
    N, ND = x.shape
    E, ED = edge_attr.shape
    B = task_id.shape[0]
    H = Wemb.shape[1]
    L = We1.shape[0]

    src2g = edge_index[0].reshape(E // 64, 64)
    dst2g = edge_index[1].reshape(E // 64, 64)
    dst2 = edge_index[1].reshape(E // 128, 128)
    p8 = jnp.pad(pos, ((0, 0), (0, 8 - pos.shape[1])))
    batch_f = batch_idx.astype(F32).reshape(N // NBLK, 1, NBLK)
    tid2 = task_id.reshape(B, 1)
    hn = N // 2
    hnp = ((hn + 64 + 127) // 128) * 128
    zeros_nw = jnp.zeros((N, 128), F32)
    zeros_x = jnp.zeros((hnp * 4,), F32)

    w1d = We1[:, :H, :]
    w1s = We1[:, H:2 * H, :]
    w1x = We1[:, 2 * H:, :]
    wh1a = Wh1[:, :H, :]
    wh1b = Wh1[:, H:, :]
    wx2r = Wx2.reshape(L, 1, H)
    whbr = Whb.reshape(Whb.shape[0], Whb.shape[1])

    gn = N // NBLK
    ge = E // EBLK

    full = lambda shape: pl.BlockSpec(shape, lambda *_: tuple(0 for _ in shape))
    rowsN = lambda w: pl.BlockSpec((NBLK, w), lambda i: (i, 0))
    rowsE = lambda w: pl.BlockSpec((EBLK, w), lambda i: (i, 0))

    h, tbl_d, tbl_s = pl.pallas_call(
        _pre_body,
        grid=(gn,),
        in_specs=[rowsN(ND), rowsN(8), full((ND, H)), full((1, H)),
                  full((H, H)), full((H, H))],
        out_specs=[rowsN(H), rowsN(TW), rowsN(TW)],
        out_shape=[jax.ShapeDtypeStruct((N, H), F32),
                   jax.ShapeDtypeStruct((N, TW), F32),
                   jax.ShapeDtypeStruct((N, TW), F32)],
    )(x, p8, Wemb, bemb.reshape(1, H), w1d[0], w1s[0])

    gather = _make_gather(E)
    scatter = _make_scatter(E, N)

    p8_cur = p8
    for l in range(L):
        gu, gx = gather(tbl_d, tbl_s, dst2g, src2g)

        m_rows, aux = pl.pallas_call(
            _edge_body,
            grid=(ge,),
            in_specs=[rowsE(128), rowsE(16), rowsE(ED), full((1 + ED, H)),
                      full((1, H)), full((H, H)), full((1, H)),
                      full((H, H)), full((1, H)), full((1, H)), full((1, 1))],
            out_specs=[rowsE(128), pl.BlockSpec((4, EBLK), lambda i: (0, i))],
            out_shape=[jax.ShapeDtypeStruct((E, 128), F32),
                       jax.ShapeDtypeStruct((4, E), F32)],
        )(gu, gx, edge_attr, w1x[l], be1[l].reshape(1, H), We2[l],
          be2[l].reshape(1, H), Wx1[l], bx1[l].reshape(1, H), wx2r[l],
          bx2[l].reshape(1, 1))

        aux_t = jnp.swapaxes(aux.reshape(4, E // 128, 128), 0, 1)
        acc_m, acc_x = scatter(m_rows, aux_t, dst2, zeros_nw, zeros_x)
        accm3 = acc_m.reshape(2, hnp, 128)
        accx3 = acc_x.reshape(2, hnp, 4)

        hb = gn // 2
        a_spec = pl.BlockSpec((1, NBLK, 128), lambda i: (i // hb, i % hb, 0))
        x_spec = pl.BlockSpec((1, NBLK, 4), lambda i: (i // hb, i % hb, 0))
        if l < L - 1:
            h, p8_cur, tbl_d, tbl_s = pl.pallas_call(
                _node_body,
                grid=(gn,),
                in_specs=[a_spec, x_spec, rowsN(H), rowsN(8),
                          full((H, H)), full((H, H)), full((1, H)),
                          full((H, H)), full((1, H)),
                          full((H, H)), full((H, H))],
                out_specs=[rowsN(H), rowsN(8), rowsN(TW), rowsN(TW)],
                out_shape=[jax.ShapeDtypeStruct((N, H), F32),
                           jax.ShapeDtypeStruct((N, 8), F32),
                           jax.ShapeDtypeStruct((N, TW), F32),
                           jax.ShapeDtypeStruct((N, TW), F32)],
            )(accm3, accx3, h, p8_cur, wh1a[l], wh1b[l],
              bh1[l].reshape(1, H), Wh2[l], bh2[l].reshape(1, H),
              w1d[l + 1], w1s[l + 1])
        else:
            h = pl.pallas_call(
                _node_last_body,
                grid=(gn,),
                in_specs=[a_spec, rowsN(H),
                          full((H, H)), full((H, H)), full((1, H)),
                          full((H, H)), full((1, H))],
                out_specs=rowsN(H),
                out_shape=jax.ShapeDtypeStruct((N, H), F32),
            )(accm3, h, wh1a[l], wh1b[l], bh1[l].reshape(1, H),
              Wh2[l], bh2[l].reshape(1, H))

    g_aug = pl.pallas_call(
        _pool_body,
        grid=(gn,),
        in_specs=[rowsN(H), pl.BlockSpec((1, 1, NBLK), lambda i: (i, 0, 0))],
        out_specs=pl.BlockSpec((B, PW), lambda i: (0, 0)),
        out_shape=jax.ShapeDtypeStruct((B, PW), F32),
    )(h, batch_f)

    logits = pl.pallas_call(
        _head_body,
        in_specs=[full((B, PW)), full((B, 1)), full(Wha.shape), full(bha.shape),
                  full(whbr.shape), full(bhb.shape)],
        out_specs=full((B, 1)),
        out_shape=jax.ShapeDtypeStruct((B, 1), F32),
    )(g_aug, tid2, Wha, bha, whbr, bhb)

    return logits


# speedup vs baseline: 1.4608x; 1.1060x over previous
import functools

import jax
import jax.numpy as jnp
from jax import lax
from jax.experimental import pallas as pl
from jax.experimental.pallas import tpu as pltpu
from jax.experimental.pallas import tpu_sc as plsc

F32 = jnp.float32
TW = 256
PW = 144
NBLK = 1000
EBLK = 640


def _silu(v):
    return v / (1.0 + jnp.exp(-v))


def _dot(a, b):
    return jnp.dot(a, b, preferred_element_type=F32)


def _pre_body(x_ref, p8_ref, wemb_ref, bemb_ref, wd_ref, ws_ref,
              h_ref, td_ref, ts_ref):
    h = _dot(x_ref[...], wemb_ref[...]) + bemb_ref[...]
    h_ref[...] = h
    p8 = p8_ref[...]
    z = jnp.zeros((h.shape[0], TW - 136), F32)
    td_ref[...] = jnp.concatenate([_dot(h, wd_ref[...]), p8, z], axis=1)
    ts_ref[...] = jnp.concatenate([_dot(h, ws_ref[...]), p8, z], axis=1)


_SEL48 = None


def _edge_body(u_ref, xp_ref, ea_ref, wex_ref, be1_ref, we2_ref, be2_ref,
               wx1_ref, bx1_ref, wx2_ref, bx2_ref, m_ref, aux_ref):
    u = u_ref[...]
    relp = xp_ref[...][:, :8]
    d2 = jnp.sum(relp * relp, axis=1, keepdims=True)
    extra = jnp.concatenate([d2, ea_ref[...]], axis=1)
    m1 = _silu(u + _dot(extra, wex_ref[...]) + be1_ref[...])
    m = _silu(_dot(m1, we2_ref[...]) + be2_ref[...])
    t1 = _silu(_dot(m, wx1_ref[...]) + bx1_ref[...])
    xw = jnp.sum(t1 * wx2_ref[...], axis=1, keepdims=True) + bx2_ref[...]
    m_ref[...] = m
    rx = relp * xw
    sel = jnp.concatenate(
        [jnp.eye(3, 8, dtype=F32), jnp.zeros((1, 8), F32)], axis=0)
    aux = lax.dot_general(sel, rx, (((1,), (1,)), ((), ())),
                          preferred_element_type=F32)
    aux_ref[...] = aux + jnp.concatenate(
        [jnp.zeros((3, aux.shape[1]), F32), jnp.ones((1, aux.shape[1]), F32)], axis=0)


def _node_body(a_ref, x_ref, h_ref, p8_ref, wh1a_ref,
               wh1b_ref, bh1_ref, wh2_ref, bh2_ref, wd_ref, ws_ref,
               hn_ref, pn_ref, td_ref, ts_ref):
    aggm = a_ref[0]
    small = x_ref[0]
    deg = small[:, 3:4]
    n = small.shape[0]
    aggx = jnp.concatenate([small[:, :3], jnp.zeros((n, 5), F32)], axis=1)
    p_new = p8_ref[...] + aggx / (deg + 1.0)
    h = h_ref[...]
    hu = _silu(_dot(h, wh1a_ref[...]) + _dot(aggm, wh1b_ref[...]) + bh1_ref[...])
    h_new = h + _dot(hu, wh2_ref[...]) + bh2_ref[...]
    hn_ref[...] = h_new
    pn_ref[...] = p_new
    if td_ref is not None:
        z = jnp.zeros((h.shape[0], TW - 136), F32)
        td_ref[...] = jnp.concatenate([_dot(h_new, wd_ref[...]), p_new, z], axis=1)
        ts_ref[...] = jnp.concatenate([_dot(h_new, ws_ref[...]), p_new, z], axis=1)


def _node_last_body(a_ref, h_ref, wh1a_ref, wh1b_ref, bh1_ref,
                    wh2_ref, bh2_ref, hn_ref):
    aggm = a_ref[0]
    h = h_ref[...]
    hu = _silu(_dot(h, wh1a_ref[...]) + _dot(aggm, wh1b_ref[...]) + bh1_ref[...])
    hn_ref[...] = h + _dot(hu, wh2_ref[...]) + bh2_ref[...]


def _pool_body(h_ref, bf_ref, g_ref):
    i = pl.program_id(0)

    @pl.when(i == 0)
    def _():
        g_ref[...] = jnp.zeros_like(g_ref)

    n = h_ref.shape[0]
    bf = bf_ref[0]
    rows = lax.broadcasted_iota(jnp.int32, (128, n), 0).astype(F32)
    onehot = jnp.where(rows == bf, 1.0, 0.0)
    hb = jnp.concatenate([h_ref[...], jnp.ones((n, 16), F32)], axis=1)
    g_ref[...] += _dot(onehot, hb)


def _head_body(g_ref, tid_ref, wha_ref, bha_ref, whb_ref, bhb_ref, out_ref):
    ga = g_ref[...]
    cnt = jnp.maximum(ga[:, 128:129], 1.0)
    g = ga[:, :128] / cnt
    tid = tid_ref[...]
    nt = wha_ref.shape[0]
    logits = jnp.zeros((g.shape[0], 1), F32)
    for t in range(nt):
        hid = _silu(_dot(g, wha_ref[t]) + bha_ref[t][None, :])
        o = jnp.sum(hid * whb_ref[t][None, :], axis=1, keepdims=True) + bhb_ref[t, 0]
        logits = jnp.where(tid == t, o, logits)
    out_ref[...] = logits


def _sc_mesh():
    return plsc.VectorSubcoreMesh(core_axis_name="c", subcore_axis_name="s")


def _make_gather(E):
    nch = E // 64
    tmax = (nch + 63) // 64
    jmax = 2 * tmax + 2

    def _compute(db, sb, ub, xb):
        @pl.loop(0, 64, unroll=8)
        def _(r):
            for c in range(8):
                sl = pl.ds(c * 16, 16)
                ub[r, sl] = db[r, sl] + sb[r, sl]
            pp = pl.ds(128, 16)
            xb[r, :] = db[r, pp] - sb[r, pp]

    @functools.partial(
        pl.kernel,
        out_type=(jax.ShapeDtypeStruct((E, 128), F32),
                  jax.ShapeDtypeStruct((E, 16), F32)),
        mesh=_sc_mesh(),
        scratch_types=[
            pltpu.VMEM((jmax, 64), jnp.int32), pltpu.VMEM((jmax, 64), jnp.int32),
            pltpu.VMEM((64, TW), F32), pltpu.VMEM((64, TW), F32),
            pltpu.VMEM((64, TW), F32), pltpu.VMEM((64, TW), F32),
            pltpu.VMEM((64, 128), F32), pltpu.VMEM((64, 128), F32),
            pltpu.VMEM((64, 16), F32), pltpu.VMEM((64, 16), F32),
            pltpu.SemaphoreType.DMA, pltpu.SemaphoreType.DMA,
            pltpu.SemaphoreType.DMA, pltpu.SemaphoreType.DMA,
            pltpu.SemaphoreType.DMA,
        ],
    )
    def gath(tbl_d, tbl_s, dst2, src2, out_u, out_x,
             dall, sall, db_a, sb_a, db_b, sb_b,
             ub_a, ub_b, xb_a, xb_b, sg_a, sg_b, sw_a, sw_b, si):
        wid = lax.axis_index("s") * 2 + lax.axis_index("c")

        @pl.loop(0, jmax)
        def _(j):
            @pl.when(wid + 32 * j < nch)
            def _():
                pltpu.async_copy(dst2.at[wid + 32 * j], dall.at[j], si)
                pltpu.async_copy(src2.at[wid + 32 * j], sall.at[j], si)

        @pl.loop(0, jmax)
        def _(j):
            @pl.when(wid + 32 * j < nch)
            def _():
                pltpu.make_async_copy(dst2.at[0], dall.at[0], si).wait()
                pltpu.make_async_copy(src2.at[0], sall.at[0], si).wait()

        def stage(j, db, sb, sg):
            @pl.when(wid + 32 * j < nch)
            def _():
                pltpu.async_copy(tbl_d.at[dall.at[j]], db, sg)
                pltpu.async_copy(tbl_s.at[sall.at[j]], sb, sg)

        def consume(j, db, sb, ub, xb, sg, sw):
            ci = wid + 32 * j

            @pl.when(ci < nch)
            def _():
                pltpu.make_async_copy(tbl_d.at[pl.ds(0, 64)], db, sg).wait()
                pltpu.make_async_copy(tbl_s.at[pl.ds(0, 64)], sb, sg).wait()

                @pl.when(j >= 2)
                def _():
                    pltpu.make_async_copy(out_u.at[pl.ds(0, 64)], ub, sw).wait()
                    pltpu.make_async_copy(out_x.at[pl.ds(0, 64)], xb, sw).wait()

                _compute(db, sb, ub, xb)
                pltpu.async_copy(ub, out_u.at[pl.ds(ci * 64, 64)], sw)
                pltpu.async_copy(xb, out_x.at[pl.ds(ci * 64, 64)], sw)

        stage(0, db_a, sb_a, sg_a)

        @pl.loop(0, tmax)
        def _(t):
            stage(2 * t + 1, db_b, sb_b, sg_b)
            consume(2 * t, db_a, sb_a, ub_a, xb_a, sg_a, sw_a)
            stage(2 * t + 2, db_a, sb_a, sg_a)
            consume(2 * t + 1, db_b, sb_b, ub_b, xb_b, sg_b, sw_b)

        pltpu.make_async_copy(out_u.at[pl.ds(0, 64)], ub_a, sw_a).wait()
        pltpu.make_async_copy(out_x.at[pl.ds(0, 64)], xb_a, sw_a).wait()
        pltpu.make_async_copy(out_u.at[pl.ds(0, 64)], ub_b, sw_b).wait()
        pltpu.make_async_copy(out_x.at[pl.ds(0, 64)], xb_b, sw_b).wait()

    return gath


def _make_scatter(E, N):
    nch = E // 128
    hn = N // 2
    hnp = ((hn + 64 + 127) // 128) * 128
    rpt = hnp // 16
    hnp4 = hnp * 4
    xsl = hnp4 // 16

    @functools.partial(
        pl.kernel,
        out_type=(jax.ShapeDtypeStruct((2 * hnp, 128), F32),
                  jax.ShapeDtypeStruct((2 * hnp4,), F32)),
        mesh=_sc_mesh(),
        scratch_types=[
            pltpu.VMEM((128,), jnp.int32), pltpu.VMEM((128,), jnp.int32),
            pltpu.VMEM((128,), jnp.int32),
            pltpu.VMEM((4, 128), jnp.int32), pltpu.VMEM((4, 128), jnp.int32),
            pltpu.VMEM((128, 128), F32), pltpu.VMEM((128, 128), F32),
            pltpu.VMEM((4, 128), F32), pltpu.VMEM((4, 128), F32),
            pltpu.VMEM((xsl,), F32),
            pltpu.SemaphoreType.DMA, pltpu.SemaphoreType.DMA,
            pltpu.SemaphoreType.DMA, pltpu.SemaphoreType.DMA,
            pltpu.VMEM_SHARED((hnp, 128), F32),
            pltpu.VMEM_SHARED((hnp4,), F32),
        ],
    )
    def scat(m_rows, aux_t, dst2, zeros_nw, zeros_x, out_m, out_x,
             di_a, di_b, mi_v, xk_a, xk_b, mb_a, mb_b, xv_a, xv_b, red_v,
             sm_a, sm_b, sx_a, sx_b, acc_sh, acx_sh):
        c0 = lax.axis_index("c")
        s0 = lax.axis_index("s")
        base = s0 * rpt
        lo = c0 * hn
        xbase = s0 * xsl

        pltpu.sync_copy(zeros_nw.at[pl.ds(base, rpt)],
                        acc_sh.at[pl.ds(base, rpt)])
        pltpu.sync_copy(zeros_x.at[pl.ds(xbase, xsl)], red_v)
        pltpu.sync_copy(red_v, acx_sh.at[pl.ds(xbase, xsl)])
        plsc.subcore_barrier()

        def stage(ci, di, mb, xv, sm):
            @pl.when(ci < nch)
            def _():
                pltpu.async_copy(dst2.at[ci], di, sm)
                pltpu.async_copy(m_rows.at[pl.ds(ci * 128, 128)], mb, sm)
                pltpu.async_copy(aux_t.at[ci], xv, sm)

        def consume(ci, di, mb, xv, xk, sm, sx):
            @pl.when(ci < nch)
            def _():
                pltpu.make_async_copy(dst2.at[0], di, sm).wait()
                pltpu.make_async_copy(m_rows.at[pl.ds(0, 128)], mb, sm).wait()
                pltpu.make_async_copy(aux_t.at[0], xv, sm).wait()

                @pl.when(ci >= s0 + 32)
                def _():
                    for k in range(4):
                        pltpu.make_async_copy(aux_t.at[0, 0], xv.at[0],
                                              sx).wait()

                for j in range(8):
                    sl = pl.ds(j * 16, 16)
                    di16 = di[sl]
                    off = di16 - lo
                    ok = (off >= 0) & (off < hn)
                    mi_v[sl] = jnp.where(ok, off, hn + (di16 & 63))
                    xb = jnp.where(ok, off * 4, hn * 4 + (di16 & 255))
                    for k in range(4):
                        xk[k, sl] = xb + k
                pltpu.sync_copy(mb, acc_sh.at[mi_v], add=True)
                for k in range(4):
                    pltpu.async_copy(xv.at[k], acx_sh.at[xk.at[k]], sx,
                                     add=True)

        stage(s0, di_a, mb_a, xv_a, sm_a)

        @pl.loop(0, (nch + 31) // 32)
        def _(t):
            ci0 = s0 + 32 * t
            ci1 = ci0 + 16
            stage(ci1, di_b, mb_b, xv_b, sm_b)
            consume(ci0, di_a, mb_a, xv_a, xk_a, sm_a, sx_a)
            stage(ci0 + 32, di_a, mb_a, xv_a, sm_a)
            consume(ci1, di_b, mb_b, xv_b, xk_b, sm_b, sx_b)

        for k in range(4):
            pltpu.make_async_copy(aux_t.at[0, 0], xv_a.at[0], sx_a).wait()
            pltpu.make_async_copy(aux_t.at[0, 0], xv_b.at[0], sx_b).wait()
        plsc.subcore_barrier()

        pltpu.sync_copy(acc_sh.at[pl.ds(base, rpt)],
                        out_m.at[pl.ds(c0 * hnp + base, rpt)])
        pltpu.sync_copy(acx_sh.at[pl.ds(xbase, xsl)], red_v)
        pltpu.sync_copy(red_v, out_x.at[pl.ds(c0 * hnp4 + xbase, xsl)])

    return scat


def kernel(x, pos, edge_attr, edge_index, batch_idx, task_id, Wemb, bemb,
           We1, be1, We2, be2, Wx1, bx1, Wx2, bx2, Wh1, bh1, Wh2, bh2,
           Wha, bha, Whb, bhb):
    N, ND = x.shape
    E, ED = edge_attr.shape
    B = task_id.shape[0]
    H = Wemb.shape[1]
    L = We1.shape[0]

    src2g = edge_index[0].reshape(E // 64, 64)
    dst2g = edge_index[1].reshape(E // 64, 64)
    dst2 = edge_index[1].reshape(E // 128, 128)
    p8 = jnp.pad(pos, ((0, 0), (0, 8 - pos.shape[1])))
    batch_f = batch_idx.astype(F32).reshape(N // NBLK, 1, NBLK)
    tid2 = task_id.reshape(B, 1)
    hn = N // 2
    hnp = ((hn + 64 + 127) // 128) * 128
    zeros_nw = jnp.zeros((N, 128), F32)
    zeros_x = jnp.zeros((hnp * 4,), F32)

    w1d = We1[:, :H, :]
    w1s = We1[:, H:2 * H, :]
    w1x = We1[:, 2 * H:, :]
    wh1a = Wh1[:, :H, :]
    wh1b = Wh1[:, H:, :]
    wx2r = Wx2.reshape(L, 1, H)
    whbr = Whb.reshape(Whb.shape[0], Whb.shape[1])

    gn = N // NBLK
    ge = E // EBLK

    full = lambda shape: pl.BlockSpec(shape, lambda *_: tuple(0 for _ in shape))
    rowsN = lambda w: pl.BlockSpec((NBLK, w), lambda i: (i, 0))
    rowsE = lambda w: pl.BlockSpec((EBLK, w), lambda i: (i, 0))

    h, tbl_d, tbl_s = pl.pallas_call(
        _pre_body,
        grid=(gn,),
        in_specs=[rowsN(ND), rowsN(8), full((ND, H)), full((1, H)),
                  full((H, H)), full((H, H))],
        out_specs=[rowsN(H), rowsN(TW), rowsN(TW)],
        out_shape=[jax.ShapeDtypeStruct((N, H), F32),
                   jax.ShapeDtypeStruct((N, TW), F32),
                   jax.ShapeDtypeStruct((N, TW), F32)],
    )(x, p8, Wemb, bemb.reshape(1, H), w1d[0], w1s[0])

    gather = _make_gather(E)
    scatter = _make_scatter(E, N)

    p8_cur = p8
    for l in range(L):
        gu, gx = gather(tbl_d, tbl_s, dst2g, src2g)

        m_rows, aux = pl.pallas_call(
            _edge_body,
            grid=(ge,),
            in_specs=[rowsE(128), rowsE(16), rowsE(ED), full((1 + ED, H)),
                      full((1, H)), full((H, H)), full((1, H)),
                      full((H, H)), full((1, H)), full((1, H)), full((1, 1))],
            out_specs=[rowsE(128), pl.BlockSpec((4, EBLK), lambda i: (0, i))],
            out_shape=[jax.ShapeDtypeStruct((E, 128), F32),
                       jax.ShapeDtypeStruct((4, E), F32)],
        )(gu, gx, edge_attr, w1x[l], be1[l].reshape(1, H), We2[l],
          be2[l].reshape(1, H), Wx1[l], bx1[l].reshape(1, H), wx2r[l],
          bx2[l].reshape(1, 1))

        aux_t = jnp.swapaxes(aux.reshape(4, E // 128, 128), 0, 1)
        acc_m, acc_x = scatter(m_rows, aux_t, dst2, zeros_nw, zeros_x)
        accm3 = acc_m.reshape(2, hnp, 128)
        accx3 = acc_x.reshape(2, hnp, 4)

        hb = gn // 2
        a_spec = pl.BlockSpec((1, NBLK, 128), lambda i: (i // hb, i % hb, 0))
        x_spec = pl.BlockSpec((1, NBLK, 4), lambda i: (i // hb, i % hb, 0))
        if l < L - 1:
            h, p8_cur, tbl_d, tbl_s = pl.pallas_call(
                _node_body,
                grid=(gn,),
                in_specs=[a_spec, x_spec, rowsN(H), rowsN(8),
                          full((H, H)), full((H, H)), full((1, H)),
                          full((H, H)), full((1, H)),
                          full((H, H)), full((H, H))],
                out_specs=[rowsN(H), rowsN(8), rowsN(TW), rowsN(TW)],
                out_shape=[jax.ShapeDtypeStruct((N, H), F32),
                           jax.ShapeDtypeStruct((N, 8), F32),
                           jax.ShapeDtypeStruct((N, TW), F32),
                           jax.ShapeDtypeStruct((N, TW), F32)],
            )(accm3, accx3, h, p8_cur, wh1a[l], wh1b[l],
              bh1[l].reshape(1, H), Wh2[l], bh2[l].reshape(1, H),
              w1d[l + 1], w1s[l + 1])
        else:
            h = pl.pallas_call(
                _node_last_body,
                grid=(gn,),
                in_specs=[a_spec, rowsN(H),
                          full((H, H)), full((H, H)), full((1, H)),
                          full((H, H)), full((1, H))],
                out_specs=rowsN(H),
                out_shape=jax.ShapeDtypeStruct((N, H), F32),
            )(accm3, h, wh1a[l], wh1b[l], bh1[l].reshape(1, H),
              Wh2[l], bh2[l].reshape(1, H))

    g_aug = pl.pallas_call(
        _pool_body,
        grid=(gn,),
        in_specs=[rowsN(H), pl.BlockSpec((1, 1, NBLK), lambda i: (i, 0, 0))],
        out_specs=pl.BlockSpec((B, PW), lambda i: (0, 0)),
        out_shape=jax.ShapeDtypeStruct((B, PW), F32),
    )(h, batch_f)

    logits = pl.pallas_call(
        _head_body,
        in_specs=[full((B, PW)), full((B, 1)), full(Wha.shape), full(bha.shape),
                  full(whbr.shape), full(bhb.shape)],
        out_specs=full((B, 1)),
        out_shape=jax.ShapeDtypeStruct((B, 1), F32),
    )(g_aug, tid2, Wha, bha, whbr, bhb)

    return logits
